# baseline (speedup 1.0000x reference)
import functools

import jax
import jax.numpy as jnp
from jax import lax
from jax.experimental import pallas as pl
from jax.experimental.pallas import tpu as pltpu
from jax.experimental.pallas import tpu_sc as plsc

VOCAB = 100000
HIDDEN = 768
PAD_IDX = 1
LN_EPS = 1e-12
LANES = 16

NC, NS = 2, 16
NW = NC * NS


HALF = HIDDEN // 2


def _sc_gather_add_body(word_hbm, pos_hbm, ids_hbm, pid_hbm, out_hbm,
                        ids_v, pid_v, bw0, bw1, bp0, bp1, pk0, pk1,
                        sw0, sw1, sp0, sp1, so0, so1,
                        *, tokens_per_worker, chunk):
    wid = lax.axis_index("s") * NC + lax.axis_index("c")
    base = wid * tokens_per_worker
    nchunks = tokens_per_worker // chunk
    npairs = nchunks // 2
    bw = (bw0, bw1)
    bp = (bp0, bp1)
    pk = (pk0, pk1)
    sw = (sw0, sw1)
    sp = (sp0, sp1)
    so = (so0, so1)

    pltpu.sync_copy(ids_hbm.at[pl.ds(base, tokens_per_worker)], ids_v)
    pltpu.sync_copy(pid_hbm.at[pl.ds(base, tokens_per_worker)], pid_v)

    def start_gathers(ci, b):
        sl = pl.ds(ci * chunk, chunk)
        pltpu.async_copy(word_hbm.at[ids_v.at[sl]], bw[b], sw[b])
        pltpu.async_copy(pos_hbm.at[pid_v.at[sl]], bp[b], sp[b])

    start_gathers(0, 0)
    start_gathers(1, 1)

    half_bias = jnp.int32(0x8000)
    hi_mask = jnp.int32(-65536)

    def pair_body_wrap(k, carry):
        for b in (0, 1):
            ci = 2 * k + b
            pltpu.make_async_copy(word_hbm.at[ids_v.at[pl.ds(0, chunk)]],
                                  bw[b], sw[b]).wait()
            pltpu.make_async_copy(pos_hbm.at[pid_v.at[pl.ds(0, chunk)]],
                                  bp[b], sp[b]).wait()
            out_sl = pl.ds(base + ci * chunk, chunk)

            @pl.when(k > 0)
            def _():
                pltpu.make_async_copy(pk[b], out_hbm.at[out_sl], so[b]).wait()

            def pack_row(r, _):
                for j in range(HALF // LANES):
                    sl_a = pl.ds(j * LANES, LANES)
                    sl_b = pl.ds(HALF + j * LANES, LANES)
                    a = bw[b][r, sl_a] + bp[b][r, sl_a]
                    c = bw[b][r, sl_b] + bp[b][r, sl_b]
                    ai = lax.bitcast_convert_type(a, jnp.int32)
                    ci32 = lax.bitcast_convert_type(c, jnp.int32)
                    lo = lax.shift_right_logical(ai + half_bias, 16)
                    hi = (ci32 + half_bias) & hi_mask
                    pk[b][r, sl_a] = lo | hi
                return 0

            lax.fori_loop(0, chunk, pack_row, 0)
            pltpu.async_copy(pk[b], out_hbm.at[out_sl], so[b])

            @pl.when(ci + 2 < nchunks)
            def _():
                start_gathers(ci + 2, b)
        return carry

    lax.fori_loop(0, npairs, pair_body_wrap, 0)

    last_sl = pl.ds(base + (nchunks - 2) * chunk, chunk)
    pltpu.make_async_copy(pk[0], out_hbm.at[last_sl], so[0]).wait()
    pltpu.make_async_copy(pk[1], out_hbm.at[last_sl], so[1]).wait()


def _make_sc_gather_add(n_tokens, chunk):
    tokens_per_worker = n_tokens // NW
    body = functools.partial(_sc_gather_add_body,
                             tokens_per_worker=tokens_per_worker, chunk=chunk)
    return pl.kernel(
        body,
        out_type=jax.ShapeDtypeStruct((n_tokens, HALF), jnp.int32),
        mesh=plsc.VectorSubcoreMesh(core_axis_name="c", subcore_axis_name="s",
                                    num_cores=NC, num_subcores=NS),
        scratch_types=[
            pltpu.VMEM((tokens_per_worker,), jnp.int32),
            pltpu.VMEM((tokens_per_worker,), jnp.int32),
            pltpu.VMEM((chunk, HIDDEN), jnp.float32),
            pltpu.VMEM((chunk, HIDDEN), jnp.float32),
            pltpu.VMEM((chunk, HIDDEN), jnp.float32),
            pltpu.VMEM((chunk, HIDDEN), jnp.float32),
            pltpu.VMEM((chunk, HALF), jnp.int32),
            pltpu.VMEM((chunk, HALF), jnp.int32),
            pltpu.SemaphoreType.DMA,
            pltpu.SemaphoreType.DMA,
            pltpu.SemaphoreType.DMA,
            pltpu.SemaphoreType.DMA,
            pltpu.SemaphoreType.DMA,
            pltpu.SemaphoreType.DMA,
        ],
    )


def _ln_first_kernel(x_ref, tvec_ref, w_ref, b_ref, o_ref):
    xi = x_ref[...]
    lo = lax.bitcast_convert_type(xi << 16, jnp.float32)
    hi = lax.bitcast_convert_type(xi & jnp.int32(-65536), jnp.float32)
    e = jnp.concatenate([lo, hi], axis=-1) + tvec_ref[...]
    mean = jnp.mean(e, axis=-1, keepdims=True)
    c = e - mean
    var = jnp.mean(c * c, axis=-1, keepdims=True)
    o_ref[...] = (c * lax.rsqrt(var + LN_EPS)) * w_ref[...] + b_ref[...]


def _ln_slice_kernel(buf_ref, x_ref, tvec_ref, w_ref, b_ref, o_ref):
    del buf_ref
    _ln_first_kernel(x_ref, tvec_ref, w_ref, b_ref, o_ref)


def _ln_slice(buf, summed_k, tvec, w, b, *, k, n_total, block_rows):
    n_k = summed_k.shape[0]
    steps = n_k // block_rows
    base = k * steps
    if buf is None:
        return pl.pallas_call(
            _ln_first_kernel,
            grid=(steps,),
            in_specs=[
                pl.BlockSpec((block_rows, HALF), lambda i: (i, 0)),
                pl.BlockSpec((1, HIDDEN), lambda i: (0, 0)),
                pl.BlockSpec((1, HIDDEN), lambda i: (0, 0)),
                pl.BlockSpec((1, HIDDEN), lambda i: (0, 0)),
            ],
            out_specs=pl.BlockSpec((block_rows, HIDDEN),
                                   lambda i, base=base: (base + i, 0)),
            out_shape=jax.ShapeDtypeStruct((n_total, HIDDEN), jnp.float32),
        )(summed_k, tvec, w, b)
    return pl.pallas_call(
        _ln_slice_kernel,
        grid=(steps,),
        in_specs=[
            pl.BlockSpec((8, HIDDEN), lambda i: (0, 0)),
            pl.BlockSpec((block_rows, HALF), lambda i: (i, 0)),
            pl.BlockSpec((1, HIDDEN), lambda i: (0, 0)),
            pl.BlockSpec((1, HIDDEN), lambda i: (0, 0)),
            pl.BlockSpec((1, HIDDEN), lambda i: (0, 0)),
        ],
        out_specs=pl.BlockSpec((block_rows, HIDDEN),
                               lambda i, base=base: (base + i, 0)),
        out_shape=jax.ShapeDtypeStruct((n_total, HIDDEN), jnp.float32),
        input_output_aliases={0: 0},
    )(buf, summed_k, tvec, w, b)


def kernel(input_ids, word_emb, pos_emb, type_emb, ln_weight, ln_bias):
    B, S = input_ids.shape
    n = B * S
    nslices = 2
    n_k = n // nslices
    ids = input_ids.reshape(-1).astype(jnp.int32)
    mask = (input_ids != PAD_IDX).astype(jnp.int32)
    pos = (jnp.cumsum(mask, axis=1) * mask + PAD_IDX).astype(jnp.int32)
    pos = pos.reshape(-1)

    gather = _make_sc_gather_add(n_k, chunk=32)
    summed = [gather(word_emb, pos_emb,
                     lax.dynamic_slice_in_dim(ids, k * n_k, n_k),
                     lax.dynamic_slice_in_dim(pos, k * n_k, n_k))
              for k in range(nslices)]

    tvec = type_emb[0].reshape(1, HIDDEN)
    w = ln_weight.reshape(1, HIDDEN)
    b = ln_bias.reshape(1, HIDDEN)
    buf = None
    for k in range(nslices):
        buf = _ln_slice(buf, summed[k], tvec, w, b,
                        k=k, n_total=n, block_rows=2048)
    return buf.reshape(B, S, HIDDEN)


# speedup vs baseline: 1.5652x; 1.5652x over previous
import functools

import jax
import jax.numpy as jnp
from jax import lax
from jax.experimental import pallas as pl
from jax.experimental.pallas import tpu as pltpu
from jax.experimental.pallas import tpu_sc as plsc

VOCAB = 100000
HIDDEN = 768
PAD_IDX = 1
LN_EPS = 1e-12
LANES = 16

NC, NS = 2, 16
NW = NC * NS


HALF = HIDDEN // 2


def _sc_gather_add_body(word_hbm, pos_hbm, ids_hbm, pid_hbm, out_hbm,
                        ids_v, pid_v, bw0, bw1, bp0, bp1, pk0, pk1,
                        sw0, sw1, sp0, sp1, so0, so1,
                        *, tokens_per_worker, chunk):
    wid = lax.axis_index("s") * NC + lax.axis_index("c")
    base = wid * tokens_per_worker
    nchunks = tokens_per_worker // chunk
    npairs = nchunks // 2
    bw = (bw0, bw1)
    bp = (bp0, bp1)
    pk = (pk0, pk1)
    sw = (sw0, sw1)
    sp = (sp0, sp1)
    so = (so0, so1)

    pltpu.sync_copy(ids_hbm.at[pl.ds(base, tokens_per_worker)], ids_v)
    pltpu.sync_copy(pid_hbm.at[pl.ds(base, tokens_per_worker)], pid_v)

    def start_gathers(ci, b):
        sl = pl.ds(ci * chunk, chunk)
        pltpu.async_copy(word_hbm.at[ids_v.at[sl]], bw[b], sw[b])
        pltpu.async_copy(pos_hbm.at[pid_v.at[sl]], bp[b], sp[b])

    start_gathers(0, 0)
    start_gathers(1, 1)

    half_bias = jnp.int32(0x8000)
    hi_mask = jnp.int32(-65536)

    def pair_body_wrap(k, carry):
        for b in (0, 1):
            ci = 2 * k + b
            pltpu.make_async_copy(word_hbm.at[ids_v.at[pl.ds(0, chunk)]],
                                  bw[b], sw[b]).wait()
            pltpu.make_async_copy(pos_hbm.at[pid_v.at[pl.ds(0, chunk)]],
                                  bp[b], sp[b]).wait()
            out_sl = pl.ds(base + ci * chunk, chunk)

            @pl.when(k > 0)
            def _():
                pltpu.make_async_copy(pk[b], out_hbm.at[out_sl], so[b]).wait()

            @plsc.parallel_loop(0, chunk, 1, unroll=2)
            def pack_row(r):
                for j in range(HALF // LANES):
                    sl_a = pl.ds(j * LANES, LANES)
                    sl_b = pl.ds(HALF + j * LANES, LANES)
                    a = bw[b][r, sl_a] + bp[b][r, sl_a]
                    c = bw[b][r, sl_b] + bp[b][r, sl_b]
                    ai = lax.bitcast_convert_type(a, jnp.int32)
                    ci32 = lax.bitcast_convert_type(c, jnp.int32)
                    lo = lax.shift_right_logical(ai + half_bias, 16)
                    hi = (ci32 + half_bias) & hi_mask
                    pk[b][r, sl_a] = lo | hi
            pltpu.async_copy(pk[b], out_hbm.at[out_sl], so[b])

            @pl.when(ci + 2 < nchunks)
            def _():
                start_gathers(ci + 2, b)
        return carry

    lax.fori_loop(0, npairs, pair_body_wrap, 0)

    last_sl = pl.ds(base + (nchunks - 2) * chunk, chunk)
    pltpu.make_async_copy(pk[0], out_hbm.at[last_sl], so[0]).wait()
    pltpu.make_async_copy(pk[1], out_hbm.at[last_sl], so[1]).wait()


def _make_sc_gather_add(n_tokens, chunk):
    tokens_per_worker = n_tokens // NW
    body = functools.partial(_sc_gather_add_body,
                             tokens_per_worker=tokens_per_worker, chunk=chunk)
    return pl.kernel(
        body,
        out_type=jax.ShapeDtypeStruct((n_tokens, HALF), jnp.int32),
        mesh=plsc.VectorSubcoreMesh(core_axis_name="c", subcore_axis_name="s",
                                    num_cores=NC, num_subcores=NS),
        scratch_types=[
            pltpu.VMEM((tokens_per_worker,), jnp.int32),
            pltpu.VMEM((tokens_per_worker,), jnp.int32),
            pltpu.VMEM((chunk, HIDDEN), jnp.float32),
            pltpu.VMEM((chunk, HIDDEN), jnp.float32),
            pltpu.VMEM((chunk, HIDDEN), jnp.float32),
            pltpu.VMEM((chunk, HIDDEN), jnp.float32),
            pltpu.VMEM((chunk, HALF), jnp.int32),
            pltpu.VMEM((chunk, HALF), jnp.int32),
            pltpu.SemaphoreType.DMA,
            pltpu.SemaphoreType.DMA,
            pltpu.SemaphoreType.DMA,
            pltpu.SemaphoreType.DMA,
            pltpu.SemaphoreType.DMA,
            pltpu.SemaphoreType.DMA,
        ],
    )


def _ln_first_kernel(x_ref, tvec_ref, w_ref, b_ref, o_ref):
    xi = x_ref[...]
    lo = lax.bitcast_convert_type(xi << 16, jnp.float32)
    hi = lax.bitcast_convert_type(xi & jnp.int32(-65536), jnp.float32)
    e = jnp.concatenate([lo, hi], axis=-1) + tvec_ref[...]
    mean = jnp.mean(e, axis=-1, keepdims=True)
    c = e - mean
    var = jnp.mean(c * c, axis=-1, keepdims=True)
    o_ref[...] = (c * lax.rsqrt(var + LN_EPS)) * w_ref[...] + b_ref[...]


def _ln_slice_kernel(buf_ref, x_ref, tvec_ref, w_ref, b_ref, o_ref):
    del buf_ref
    _ln_first_kernel(x_ref, tvec_ref, w_ref, b_ref, o_ref)


def _ln_slice(buf, summed_k, tvec, w, b, *, k, n_total, block_rows):
    n_k = summed_k.shape[0]
    steps = n_k // block_rows
    base = k * steps
    if buf is None:
        return pl.pallas_call(
            _ln_first_kernel,
            grid=(steps,),
            in_specs=[
                pl.BlockSpec((block_rows, HALF), lambda i: (i, 0)),
                pl.BlockSpec((1, HIDDEN), lambda i: (0, 0)),
                pl.BlockSpec((1, HIDDEN), lambda i: (0, 0)),
                pl.BlockSpec((1, HIDDEN), lambda i: (0, 0)),
            ],
            out_specs=pl.BlockSpec((block_rows, HIDDEN),
                                   lambda i, base=base: (base + i, 0)),
            out_shape=jax.ShapeDtypeStruct((n_total, HIDDEN), jnp.float32),
        )(summed_k, tvec, w, b)
    return pl.pallas_call(
        _ln_slice_kernel,
        grid=(steps,),
        in_specs=[
            pl.BlockSpec((8, HIDDEN), lambda i: (0, 0)),
            pl.BlockSpec((block_rows, HALF), lambda i: (i, 0)),
            pl.BlockSpec((1, HIDDEN), lambda i: (0, 0)),
            pl.BlockSpec((1, HIDDEN), lambda i: (0, 0)),
            pl.BlockSpec((1, HIDDEN), lambda i: (0, 0)),
        ],
        out_specs=pl.BlockSpec((block_rows, HIDDEN),
                               lambda i, base=base: (base + i, 0)),
        out_shape=jax.ShapeDtypeStruct((n_total, HIDDEN), jnp.float32),
        input_output_aliases={0: 0},
    )(buf, summed_k, tvec, w, b)


def kernel(input_ids, word_emb, pos_emb, type_emb, ln_weight, ln_bias):
    B, S = input_ids.shape
    n = B * S
    nslices = 2
    n_k = n // nslices
    ids = input_ids.reshape(-1).astype(jnp.int32)
    mask = (input_ids != PAD_IDX).astype(jnp.int32)
    pos = (jnp.cumsum(mask, axis=1) * mask + PAD_IDX).astype(jnp.int32)
    pos = pos.reshape(-1)

    gather = _make_sc_gather_add(n_k, chunk=32)
    summed = [gather(word_emb, pos_emb,
                     lax.dynamic_slice_in_dim(ids, k * n_k, n_k),
                     lax.dynamic_slice_in_dim(pos, k * n_k, n_k))
              for k in range(nslices)]

    tvec = type_emb[0].reshape(1, HIDDEN)
    w = ln_weight.reshape(1, HIDDEN)
    b = ln_bias.reshape(1, HIDDEN)
    buf = None
    for k in range(nslices):
        buf = _ln_slice(buf, summed[k], tvec, w, b,
                        k=k, n_total=n, block_rows=2048)
    return buf.reshape(B, S, HIDDEN)


# speedup vs baseline: 1.6013x; 1.0230x over previous
import functools

import jax
import jax.numpy as jnp
from jax import lax
from jax.experimental import pallas as pl
from jax.experimental.pallas import tpu as pltpu
from jax.experimental.pallas import tpu_sc as plsc

VOCAB = 100000
HIDDEN = 768
PAD_IDX = 1
LN_EPS = 1e-12
LANES = 16

NC, NS = 2, 16
NW = NC * NS


HALF = HIDDEN // 2


def _sc_gather_add_body(word_hbm, pos_hbm, ids_hbm, pid_hbm, out_hbm,
                        ids_v, pid_v, bw0, bw1, bp0, bp1, pk0, pk1,
                        sw0, sw1, sp0, sp1, so0, so1,
                        *, tokens_per_worker, chunk):
    wid = lax.axis_index("s") * NC + lax.axis_index("c")
    base = wid * tokens_per_worker
    nchunks = tokens_per_worker // chunk
    npairs = nchunks // 2
    bw = (bw0, bw1)
    bp = (bp0, bp1)
    pk = (pk0, pk1)
    sw = (sw0, sw1)
    sp = (sp0, sp1)
    so = (so0, so1)

    pltpu.sync_copy(ids_hbm.at[pl.ds(base, tokens_per_worker)], ids_v)
    pltpu.sync_copy(pid_hbm.at[pl.ds(base, tokens_per_worker)], pid_v)

    def start_gathers(ci, b):
        sl = pl.ds(ci * chunk, chunk)
        pltpu.async_copy(word_hbm.at[ids_v.at[sl]], bw[b], sw[b])
        pltpu.async_copy(pos_hbm.at[pid_v.at[sl]], bp[b], sp[b])

    start_gathers(0, 0)
    start_gathers(1, 1)

    half_bias = jnp.int32(0x8000)
    hi_mask = jnp.int32(-65536)

    def pair_body_wrap(k, carry):
        for b in (0, 1):
            ci = 2 * k + b
            pltpu.make_async_copy(word_hbm.at[ids_v.at[pl.ds(0, chunk)]],
                                  bw[b], sw[b]).wait()
            pltpu.make_async_copy(pos_hbm.at[pid_v.at[pl.ds(0, chunk)]],
                                  bp[b], sp[b]).wait()
            out_sl = pl.ds(base + ci * chunk, chunk)

            @pl.when(k > 0)
            def _():
                pltpu.make_async_copy(pk[b], out_hbm.at[out_sl], so[b]).wait()

            @plsc.parallel_loop(0, chunk, 1, unroll=4)
            def pack_row(r):
                for j in range(HALF // LANES):
                    sl_a = pl.ds(j * LANES, LANES)
                    sl_b = pl.ds(HALF + j * LANES, LANES)
                    a = bw[b][r, sl_a] + bp[b][r, sl_a]
                    c = bw[b][r, sl_b] + bp[b][r, sl_b]
                    ai = lax.bitcast_convert_type(a, jnp.int32)
                    ci32 = lax.bitcast_convert_type(c, jnp.int32)
                    lo = lax.shift_right_logical(ai + half_bias, 16)
                    hi = (ci32 + half_bias) & hi_mask
                    pk[b][r, sl_a] = lo | hi
            pltpu.async_copy(pk[b], out_hbm.at[out_sl], so[b])

            @pl.when(ci + 2 < nchunks)
            def _():
                start_gathers(ci + 2, b)
        return carry

    lax.fori_loop(0, npairs, pair_body_wrap, 0)

    last_sl = pl.ds(base + (nchunks - 2) * chunk, chunk)
    pltpu.make_async_copy(pk[0], out_hbm.at[last_sl], so[0]).wait()
    pltpu.make_async_copy(pk[1], out_hbm.at[last_sl], so[1]).wait()


def _make_sc_gather_add(n_tokens, chunk):
    tokens_per_worker = n_tokens // NW
    body = functools.partial(_sc_gather_add_body,
                             tokens_per_worker=tokens_per_worker, chunk=chunk)
    return pl.kernel(
        body,
        out_type=jax.ShapeDtypeStruct((n_tokens, HALF), jnp.int32),
        mesh=plsc.VectorSubcoreMesh(core_axis_name="c", subcore_axis_name="s",
                                    num_cores=NC, num_subcores=NS),
        scratch_types=[
            pltpu.VMEM((tokens_per_worker,), jnp.int32),
            pltpu.VMEM((tokens_per_worker,), jnp.int32),
            pltpu.VMEM((chunk, HIDDEN), jnp.float32),
            pltpu.VMEM((chunk, HIDDEN), jnp.float32),
            pltpu.VMEM((chunk, HIDDEN), jnp.float32),
            pltpu.VMEM((chunk, HIDDEN), jnp.float32),
            pltpu.VMEM((chunk, HALF), jnp.int32),
            pltpu.VMEM((chunk, HALF), jnp.int32),
            pltpu.SemaphoreType.DMA,
            pltpu.SemaphoreType.DMA,
            pltpu.SemaphoreType.DMA,
            pltpu.SemaphoreType.DMA,
            pltpu.SemaphoreType.DMA,
            pltpu.SemaphoreType.DMA,
        ],
    )


def _ln_first_kernel(x_ref, tvec_ref, w_ref, b_ref, o_ref):
    xi = x_ref[...]
    lo = lax.bitcast_convert_type(xi << 16, jnp.float32)
    hi = lax.bitcast_convert_type(xi & jnp.int32(-65536), jnp.float32)
    e = jnp.concatenate([lo, hi], axis=-1) + tvec_ref[...]
    mean = jnp.mean(e, axis=-1, keepdims=True)
    c = e - mean
    var = jnp.mean(c * c, axis=-1, keepdims=True)
    o_ref[...] = (c * lax.rsqrt(var + LN_EPS)) * w_ref[...] + b_ref[...]


def _ln_slice_kernel(buf_ref, x_ref, tvec_ref, w_ref, b_ref, o_ref):
    del buf_ref
    _ln_first_kernel(x_ref, tvec_ref, w_ref, b_ref, o_ref)


def _ln_slice(buf, summed_k, tvec, w, b, *, k, n_total, block_rows):
    n_k = summed_k.shape[0]
    steps = n_k // block_rows
    base = k * steps
    if buf is None:
        return pl.pallas_call(
            _ln_first_kernel,
            grid=(steps,),
            in_specs=[
                pl.BlockSpec((block_rows, HALF), lambda i: (i, 0)),
                pl.BlockSpec((1, HIDDEN), lambda i: (0, 0)),
                pl.BlockSpec((1, HIDDEN), lambda i: (0, 0)),
                pl.BlockSpec((1, HIDDEN), lambda i: (0, 0)),
            ],
            out_specs=pl.BlockSpec((block_rows, HIDDEN),
                                   lambda i, base=base: (base + i, 0)),
            out_shape=jax.ShapeDtypeStruct((n_total, HIDDEN), jnp.float32),
        )(summed_k, tvec, w, b)
    return pl.pallas_call(
        _ln_slice_kernel,
        grid=(steps,),
        in_specs=[
            pl.BlockSpec((8, HIDDEN), lambda i: (0, 0)),
            pl.BlockSpec((block_rows, HALF), lambda i: (i, 0)),
            pl.BlockSpec((1, HIDDEN), lambda i: (0, 0)),
            pl.BlockSpec((1, HIDDEN), lambda i: (0, 0)),
            pl.BlockSpec((1, HIDDEN), lambda i: (0, 0)),
        ],
        out_specs=pl.BlockSpec((block_rows, HIDDEN),
                               lambda i, base=base: (base + i, 0)),
        out_shape=jax.ShapeDtypeStruct((n_total, HIDDEN), jnp.float32),
        input_output_aliases={0: 0},
    )(buf, summed_k, tvec, w, b)


def kernel(input_ids, word_emb, pos_emb, type_emb, ln_weight, ln_bias):
    B, S = input_ids.shape
    n = B * S
    nslices = 2
    n_k = n // nslices
    ids = input_ids.reshape(-1).astype(jnp.int32)
    mask = (input_ids != PAD_IDX).astype(jnp.int32)
    pos = (jnp.cumsum(mask, axis=1) * mask + PAD_IDX).astype(jnp.int32)
    pos = pos.reshape(-1)

    gather = _make_sc_gather_add(n_k, chunk=32)
    summed = [gather(word_emb, pos_emb,
                     lax.dynamic_slice_in_dim(ids, k * n_k, n_k),
                     lax.dynamic_slice_in_dim(pos, k * n_k, n_k))
              for k in range(nslices)]

    tvec = type_emb[0].reshape(1, HIDDEN)
    w = ln_weight.reshape(1, HIDDEN)
    b = ln_bias.reshape(1, HIDDEN)
    buf = None
    for k in range(nslices):
        buf = _ln_slice(buf, summed[k], tvec, w, b,
                        k=k, n_total=n, block_rows=2048)
    return buf.reshape(B, S, HIDDEN)
